# baseline (speedup 1.0000x reference)
import functools

import jax
import jax.numpy as jnp
from jax import lax
from jax.experimental import pallas as pl
from jax.experimental.pallas import tpu as pltpu
from jax.experimental.pallas import tpu_sc as plsc

H = 768
E = 8
K = 2
INTER = 2048
IT = 512
ONE_BITS = 0x3F800000


def _cumsum_lanes(a):
    r, c = a.shape
    s = 1
    while s < c:
        shifted = jnp.concatenate(
            [jnp.zeros((r, s), a.dtype), a[:, : c - s]], axis=1)
        a = a + shifted
        s *= 2
    return a


def _routing_kernel(cap, cstride, x_ref, wg1x_ref, wg1s_ref, bg1_ref,
                    wg2_ref, pos_ref, wcol_ref, tid_ref):
    T = x_ref.shape[0]
    x = x_ref[...]
    n = jnp.float32(H)
    s1 = jnp.sum(x, axis=1, keepdims=True)
    mean = s1 / n
    xc = x - mean
    var = jnp.sum(xc * xc, axis=1, keepdims=True) / (n - 1.0)
    std = jnp.sqrt(var)
    mn = jnp.min(x, axis=1, keepdims=True)
    mx = jnp.max(x, axis=1, keepdims=True)
    l2 = jnp.sqrt(jnp.sum(x * x, axis=1, keepdims=True))
    sp = jnp.sum((jnp.abs(x) < 1e-6).astype(jnp.float32), axis=1,
                 keepdims=True) / n
    stats = jnp.concatenate(
        [mean, std, mn, mx, l2, sp, jnp.zeros((T, 2), jnp.float32)], axis=1)

    h = (lax.dot_general(wg1x_ref[...], x, (((1,), (1,)), ((), ())),
                         preferred_element_type=jnp.float32)
         + lax.dot_general(wg1s_ref[...], stats, (((1,), (1,)), ((), ())),
                           preferred_element_type=jnp.float32)
         + bg1_ref[...])
    h = 0.5 * h * (1.0 + lax.erf(h * 0.7071067811865476))
    logits = lax.dot_general(wg2_ref[...], h, (((1,), (0,)), ((), ())),
                             preferred_element_type=jnp.float32)

    m = jnp.max(logits, axis=0, keepdims=True)
    ex = jnp.exp(logits - m)
    p = ex / jnp.sum(ex, axis=0, keepdims=True)

    iota8 = lax.broadcasted_iota(jnp.int32, (E, T), 0)
    a1 = jnp.max(p, axis=0, keepdims=True)
    e1 = jnp.min(jnp.where(p == a1, iota8, E + 1), axis=0, keepdims=True)
    pmask = jnp.where(iota8 == e1, -jnp.inf, p)
    a2 = jnp.max(pmask, axis=0, keepdims=True)
    e2 = jnp.min(jnp.where(pmask == a2, iota8, E + 1), axis=0, keepdims=True)
    wsum = a1 + a2
    w1 = a1 / wsum
    w2 = a2 / wsum

    wfull = jnp.where(iota8 == e1, w1, jnp.where(iota8 == e2, w2, 0.0))
    valid = ((iota8 == e1) | (iota8 == e2)) & (wfull > 0.0)
    wbits = jnp.where(valid, lax.bitcast_convert_type(wfull, jnp.int32),
                      jnp.int32(-1))

    lo = jnp.zeros((E, 1), jnp.int32)
    hi = jnp.full((E, 1), ONE_BITS, jnp.int32)
    for _ in range(31):
        mid = (lo + hi) // 2
        cnt = jnp.sum((wbits > mid).astype(jnp.int32), axis=1, keepdims=True)
        small = cnt < cap
        upd = lo < hi
        hi = jnp.where(upd & small, mid, hi)
        lo = jnp.where(upd & (~small), mid + 1, lo)
    mstar = lo

    gt = wbits > mstar
    eq = wbits == mstar
    n_gt = jnp.sum(gt.astype(jnp.int32), axis=1, keepdims=True)
    quota = cap - n_gt
    eq_i = eq.astype(jnp.int32)
    eq_excl = _cumsum_lanes(eq_i) - eq_i
    keep = gt | (eq & (eq_excl < quota))
    keep_i = keep.astype(jnp.int32)
    slot = _cumsum_lanes(keep_i) - keep_i
    wkeep = jnp.where(keep, wfull, 0.0)

    pos_full = jnp.where(keep, iota8 * cstride + slot,
                         iota8 * cstride + cap)
    sel1 = iota8 == e1
    sel2 = iota8 == e2
    pos0 = jnp.sum(jnp.where(sel1, pos_full, 0), axis=0, keepdims=True)
    pos1 = jnp.sum(jnp.where(sel2, pos_full, 0), axis=0, keepdims=True)
    zi = jnp.zeros((E - 2, T), jnp.int32)
    pos_ref[...] = jnp.concatenate([pos0, pos1, zi], axis=0)

    iota_c = lax.broadcasted_iota(jnp.int32, (cstride, T), 0)
    iota_tf = lax.broadcasted_iota(jnp.int32, (1, T), 1).astype(jnp.float32)
    cols = []
    tids = []
    for e in range(E):
        oh = ((iota_c == slot[e:e + 1, :]) & keep[e:e + 1, :]) \
            .astype(jnp.float32)
        rhs = jnp.concatenate([wkeep[e:e + 1, :], iota_tf], axis=0)
        r = lax.dot_general(oh, rhs, (((1,), (1,)), ((), ())),
                            preferred_element_type=jnp.float32)
        cols.append(r[:, 0:1])
        tids.append(r[:, 1:2])
    wcol_ref[...] = jnp.concatenate(cols, axis=1)
    tid_ref[...] = jnp.concatenate(tids, axis=1).astype(jnp.int32)


NC = 2
NS = 16
NW = NC * NS


def _sc_dispatch(x2d, tok_pad, n_pad):
    _, h = x2d.shape
    spw = n_pad // NW
    ch = spw // 3
    mesh = plsc.VectorSubcoreMesh(core_axis_name="c", subcore_axis_name="s")

    @functools.partial(
        pl.kernel, mesh=mesh,
        out_type=jax.ShapeDtypeStruct((n_pad, h), jnp.float32),
        scratch_types=[
            pltpu.VMEM((spw,), jnp.int32),
            pltpu.VMEM((ch, h), jnp.float32),
            pltpu.VMEM((ch, h), jnp.float32),
            pltpu.VMEM((ch, h), jnp.float32),
            pltpu.SemaphoreType.DMA,
            pltpu.SemaphoreType.DMA,
            pltpu.SemaphoreType.DMA,
            pltpu.SemaphoreType.DMA,
        ],
    )
    def k(x_hbm, tok_hbm, xbuf_hbm, idx_v, b0, b1, b2, s0, s1, s2, sw):
        wid = lax.axis_index("s") * NC + lax.axis_index("c")
        base = wid * spw
        pltpu.sync_copy(tok_hbm.at[pl.ds(base, spw)], idx_v)
        bufs = (b0, b1, b2)
        sems = (s0, s1, s2)
        nq = 7
        sub = ch // nq
        gets = []
        for c in range(3):
            for q in range(nq):
                gets.append(pltpu.async_copy(
                    x_hbm.at[idx_v.at[pl.ds(c * ch + q * sub, sub)]],
                    bufs[c].at[pl.ds(q * sub, sub)], sems[c]))
        puts = []
        for c in range(3):
            for q in range(nq):
                gets[c * nq + q].wait()
            puts.append(pltpu.async_copy(
                bufs[c], xbuf_hbm.at[pl.ds(base + c * ch, ch)], sw))
        for p in puts:
            p.wait()

    return k(x2d, tok_pad)


def _sc_combine(ybuf, pos2, T):
    h = ybuf.shape[1]
    tpw = T // NW
    ch = tpw // 4
    mesh = plsc.VectorSubcoreMesh(core_axis_name="c", subcore_axis_name="s")

    @functools.partial(
        pl.kernel, mesh=mesh,
        out_type=jax.ShapeDtypeStruct((T, h), jnp.float32),
        scratch_types=[
            pltpu.VMEM((2, tpw), jnp.int32),
            pltpu.VMEM((ch, h), jnp.float32),
            pltpu.VMEM((ch, h), jnp.float32),
            pltpu.VMEM((ch, h), jnp.float32),
            pltpu.VMEM((ch, h), jnp.float32),
            pltpu.SemaphoreType.DMA,
            pltpu.SemaphoreType.DMA,
        ],
    )
    def k(ybuf_hbm, pos_hbm, out_hbm, idx_v, a0, b0, a1, b1, sp0, sp1):
        wid = lax.axis_index("s") * NC + lax.axis_index("c")
        base = wid * tpw
        pltpu.sync_copy(pos_hbm.at[:, pl.ds(base, tpw)], idx_v)
        accs = (a0, a1)
        rows = (b0, b1)
        sems = (sp0, sp1)

        sub = ch // 2

        def fire(c):
            p = c % 2
            cps = []
            for dst in (accs[p], rows[p]):
                kk = 0 if dst is accs[p] else 1
                for q in range(2):
                    cps.append(pltpu.async_copy(
                        ybuf_hbm.at[idx_v.at[kk, pl.ds(c * ch + q * sub,
                                                       sub)]],
                        dst.at[pl.ds(q * sub, sub)], sems[p]))
            return cps

        pend = fire(0)
        for c in range(4):
            nxt = fire(c + 1) if c < 3 else None
            for cp in pend:
                cp.wait()
            p = c % 2
            acc_v = accs[p]
            rows_v = rows[p]

            def body(tt, carry):
                for j in range(h // 16):
                    sl = pl.ds(j * 16, 16)
                    acc_v[tt, sl] = acc_v[tt, sl] + rows_v[tt, sl]
                return carry

            lax.fori_loop(0, ch, body, 0)
            pltpu.sync_copy(acc_v, out_hbm.at[pl.ds(base + c * ch, ch)])
            pend = nxt

    return k(ybuf, pos2)


def _ffn_kernel(n_it, xbuf_ref, w1a_ref, w1b_ref, w2_ref, wcol_ref,
                ybuf_ref, acc_ref):
    e = pl.program_id(0)
    i = pl.program_id(1)
    xe = xbuf_ref[...]
    hg = lax.dot_general(xe, w1a_ref[0], (((1,), (1,)), ((), ())),
                         preferred_element_type=jnp.float32)
    hu = lax.dot_general(xe, w1b_ref[0], (((1,), (1,)), ((), ())),
                         preferred_element_type=jnp.float32)
    g = hg * jax.nn.sigmoid(hg) * hu
    contrib = lax.dot_general(g, w2_ref[0], (((1,), (1,)), ((), ())),
                              preferred_element_type=jnp.float32)

    @pl.when(i == 0)
    def _():
        acc_ref[...] = contrib

    @pl.when(i > 0)
    def _():
        acc_ref[...] = acc_ref[...] + contrib

    @pl.when(i == n_it - 1)
    def _():
        cstride = xbuf_ref.shape[0]
        iota_e = lax.broadcasted_iota(jnp.int32, (cstride, E), 1)
        wcolv = jnp.sum(jnp.where(iota_e == e, wcol_ref[...], 0.0),
                        axis=1, keepdims=True)
        ybuf_ref[...] = jnp.where(wcolv > 0.0, acc_ref[...] * wcolv, 0.0)


def _run_ffn(xbuf, w1, w2, wcol, cstride):
    n_it = INTER // IT
    return pl.pallas_call(
        functools.partial(_ffn_kernel, n_it),
        grid=(E, n_it),
        in_specs=[
            pl.BlockSpec((cstride, H), lambda e, i: (e, 0)),
            pl.BlockSpec((1, IT, H), lambda e, i: (e, i, 0)),
            pl.BlockSpec((1, IT, H), lambda e, i: (e, i + INTER // IT, 0)),
            pl.BlockSpec((1, H, IT), lambda e, i: (e, 0, i)),
            pl.BlockSpec((cstride, E), lambda e, i: (0, 0)),
        ],
        out_specs=pl.BlockSpec((cstride, H), lambda e, i: (e, 0)),
        out_shape=jax.ShapeDtypeStruct((E * cstride, H), jnp.float32),
        scratch_shapes=[pltpu.VMEM((cstride, H), jnp.float32)],
    )(xbuf, w1, w1, w2, wcol)


def kernel(x, t, Wg1, bg1, Wg2, Wh1, Wh2, Wl1, Wl2, snr_threshold=0.5):
    B, N, C = x.shape
    T = B * N
    cap = int(1.25 * T / E)
    cstride = ((cap + 8) + 7) // 8 * 8

    tokens = x.reshape(T, C)
    wg1x = Wg1[:, :H]
    wg1s = jnp.pad(Wg1[:, H:], ((0, 0), (0, 2)))
    bg1c = bg1.reshape(H // 2, 1)

    pos, wcol, tid = pl.pallas_call(
        functools.partial(_routing_kernel, cap, cstride),
        out_shape=(
            jax.ShapeDtypeStruct((E, T), jnp.int32),
            jax.ShapeDtypeStruct((cstride, E), jnp.float32),
            jax.ShapeDtypeStruct((cstride, E), jnp.int32),
        ),
    )(tokens, wg1x, wg1s, bg1c, Wg2)

    pos2 = pos[:2]
    n_pad = (E * cstride + NW * 8 - 1) // (NW * 8) * (NW * 8)
    tok_pad = jnp.pad(tid.T.reshape(-1), (0, n_pad - E * cstride))
    xbuf = _sc_dispatch(tokens, tok_pad, n_pad)

    use_low = t.mean() >= snr_threshold
    ybuf = lax.cond(
        use_low,
        lambda xb, wc: _run_ffn(xb, Wl1, Wl2, wc, cstride),
        lambda xb, wc: _run_ffn(xb, Wh1, Wh2, wc, cstride),
        xbuf, wcol)

    out = _sc_combine(ybuf, pos2, T)

    y = out.reshape(B, N, C)
    return (y, jnp.zeros((), dtype=jnp.float32))


# speedup vs baseline: 1.1950x; 1.1950x over previous
import functools

import jax
import jax.numpy as jnp
from jax import lax
from jax.experimental import pallas as pl
from jax.experimental.pallas import tpu as pltpu
from jax.experimental.pallas import tpu_sc as plsc

H = 768
E = 8
K = 2
INTER = 2048
IT = 512
ONE_BITS = 0x3F800000


def _cumsum_lanes(a):
    r, c = a.shape
    s = 1
    while s < c:
        shifted = jnp.concatenate(
            [jnp.zeros((r, s), a.dtype), a[:, : c - s]], axis=1)
        a = a + shifted
        s *= 2
    return a


def _routing_kernel(cap, cstride, x_ref, wg1x_ref, wg1s_ref, bg1_ref,
                    wg2_ref, wcol_ref, tid_ref):
    T = x_ref.shape[0]
    x = x_ref[...]
    n = jnp.float32(H)
    s1 = jnp.sum(x, axis=1, keepdims=True)
    mean = s1 / n
    xc = x - mean
    var = jnp.sum(xc * xc, axis=1, keepdims=True) / (n - 1.0)
    std = jnp.sqrt(var)
    mn = jnp.min(x, axis=1, keepdims=True)
    mx = jnp.max(x, axis=1, keepdims=True)
    l2 = jnp.sqrt(jnp.sum(x * x, axis=1, keepdims=True))
    sp = jnp.sum((jnp.abs(x) < 1e-6).astype(jnp.float32), axis=1,
                 keepdims=True) / n
    stats = jnp.concatenate(
        [mean, std, mn, mx, l2, sp, jnp.zeros((T, 2), jnp.float32)], axis=1)

    h = (lax.dot_general(wg1x_ref[...], x, (((1,), (1,)), ((), ())),
                         preferred_element_type=jnp.float32)
         + lax.dot_general(wg1s_ref[...], stats, (((1,), (1,)), ((), ())),
                           preferred_element_type=jnp.float32)
         + bg1_ref[...])
    h = 0.5 * h * (1.0 + lax.erf(h * 0.7071067811865476))
    logits = lax.dot_general(wg2_ref[...], h, (((1,), (0,)), ((), ())),
                             preferred_element_type=jnp.float32)

    m = jnp.max(logits, axis=0, keepdims=True)
    ex = jnp.exp(logits - m)
    p = ex / jnp.sum(ex, axis=0, keepdims=True)

    iota8 = lax.broadcasted_iota(jnp.int32, (E, T), 0)
    a1 = jnp.max(p, axis=0, keepdims=True)
    e1 = jnp.min(jnp.where(p == a1, iota8, E + 1), axis=0, keepdims=True)
    pmask = jnp.where(iota8 == e1, -jnp.inf, p)
    a2 = jnp.max(pmask, axis=0, keepdims=True)
    e2 = jnp.min(jnp.where(pmask == a2, iota8, E + 1), axis=0, keepdims=True)
    wsum = a1 + a2
    w1 = a1 / wsum
    w2 = a2 / wsum

    wfull = jnp.where(iota8 == e1, w1, jnp.where(iota8 == e2, w2, 0.0))
    valid = ((iota8 == e1) | (iota8 == e2)) & (wfull > 0.0)
    wbits = jnp.where(valid, lax.bitcast_convert_type(wfull, jnp.int32),
                      jnp.int32(-1))

    lo = jnp.zeros((E, 1), jnp.int32)
    hi = jnp.full((E, 1), ONE_BITS, jnp.int32)
    for _ in range(31):
        mid = (lo + hi) // 2
        cnt = jnp.sum((wbits > mid).astype(jnp.int32), axis=1, keepdims=True)
        small = cnt < cap
        upd = lo < hi
        hi = jnp.where(upd & small, mid, hi)
        lo = jnp.where(upd & (~small), mid + 1, lo)
    mstar = lo

    gt = wbits > mstar
    eq = wbits == mstar
    n_gt = jnp.sum(gt.astype(jnp.int32), axis=1, keepdims=True)
    quota = cap - n_gt
    eq_i = eq.astype(jnp.int32)
    eq_excl = _cumsum_lanes(eq_i) - eq_i
    keep = gt | (eq & (eq_excl < quota))
    keep_i = keep.astype(jnp.int32)
    slot = _cumsum_lanes(keep_i) - keep_i
    wkeep = jnp.where(keep, wfull, 0.0)

    iota_c = lax.broadcasted_iota(jnp.int32, (cstride, T), 0)
    iota_tf = lax.broadcasted_iota(jnp.int32, (1, T), 1).astype(jnp.float32)
    cols = []
    tids = []
    for e in range(E):
        oh = ((iota_c == slot[e:e + 1, :]) & keep[e:e + 1, :]) \
            .astype(jnp.float32)
        rhs = jnp.concatenate([wkeep[e:e + 1, :], iota_tf], axis=0)
        r = lax.dot_general(oh, rhs, (((1,), (1,)), ((), ())),
                            preferred_element_type=jnp.float32)
        cols.append(r[:, 0:1])
        tids.append(r[:, 1:2])
    wcol_ref[...] = jnp.concatenate(cols, axis=1)
    tid_ref[...] = jnp.concatenate(tids, axis=1).astype(jnp.int32)


NC = 2
NS = 16
NW = NC * NS


def _sc_dispatch(x2d, tok_pad, n_pad):
    _, h = x2d.shape
    spw = n_pad // NW
    ch = spw // 3
    mesh = plsc.VectorSubcoreMesh(core_axis_name="c", subcore_axis_name="s")

    @functools.partial(
        pl.kernel, mesh=mesh,
        out_type=jax.ShapeDtypeStruct((n_pad, h), jnp.float32),
        scratch_types=[
            pltpu.VMEM((spw,), jnp.int32),
            pltpu.VMEM((ch, h), jnp.float32),
            pltpu.VMEM((ch, h), jnp.float32),
            pltpu.VMEM((ch, h), jnp.float32),
            pltpu.SemaphoreType.DMA,
            pltpu.SemaphoreType.DMA,
            pltpu.SemaphoreType.DMA,
            pltpu.SemaphoreType.DMA,
        ],
    )
    def k(x_hbm, tok_hbm, xbuf_hbm, idx_v, b0, b1, b2, s0, s1, s2, sw):
        wid = lax.axis_index("s") * NC + lax.axis_index("c")
        base = wid * spw
        pltpu.sync_copy(tok_hbm.at[pl.ds(base, spw)], idx_v)
        bufs = (b0, b1, b2)
        sems = (s0, s1, s2)
        nq = 7
        sub = ch // nq
        gets = []
        for c in range(3):
            for q in range(nq):
                gets.append(pltpu.async_copy(
                    x_hbm.at[idx_v.at[pl.ds(c * ch + q * sub, sub)]],
                    bufs[c].at[pl.ds(q * sub, sub)], sems[c]))
        puts = []
        for c in range(3):
            for q in range(nq):
                gets[c * nq + q].wait()
            puts.append(pltpu.async_copy(
                bufs[c], xbuf_hbm.at[pl.ds(base + c * ch, ch)], sw))
        for p in puts:
            p.wait()

    return k(x2d, tok_pad)


def _combine_kernel(ybuf_ref, tid_ref, wcol_ref, out_ref):
    e = pl.program_id(0)
    cstride = ybuf_ref.shape[0]
    T = out_ref.shape[0]
    iota_cols = lax.broadcasted_iota(jnp.int32, (cstride, E), 1)
    tcol = jnp.sum(jnp.where(iota_cols == e, tid_ref[...], 0), axis=1,
                   keepdims=True)
    wcolv = jnp.sum(jnp.where(iota_cols == e, wcol_ref[...], 0.0), axis=1,
                    keepdims=True)
    iota_t = lax.broadcasted_iota(jnp.int32, (cstride, T), 1)
    m = ((tcol == iota_t) & (wcolv > 0.0)).astype(jnp.float32)
    contrib = lax.dot_general(m, ybuf_ref[...], (((0,), (0,)), ((), ())),
                              preferred_element_type=jnp.float32)

    @pl.when(e == 0)
    def _():
        out_ref[...] = contrib

    @pl.when(e > 0)
    def _():
        out_ref[...] = out_ref[...] + contrib


def _ffn_kernel(n_it, xbuf_ref, w1a_ref, w1b_ref, w2_ref, wcol_ref,
                ybuf_ref, acc_ref):
    e = pl.program_id(0)
    i = pl.program_id(1)
    xe = xbuf_ref[...]
    hg = lax.dot_general(xe, w1a_ref[0], (((1,), (1,)), ((), ())),
                         preferred_element_type=jnp.float32)
    hu = lax.dot_general(xe, w1b_ref[0], (((1,), (1,)), ((), ())),
                         preferred_element_type=jnp.float32)
    g = hg * jax.nn.sigmoid(hg) * hu
    contrib = lax.dot_general(g, w2_ref[0], (((1,), (1,)), ((), ())),
                              preferred_element_type=jnp.float32)

    @pl.when(i == 0)
    def _():
        acc_ref[...] = contrib

    @pl.when(i > 0)
    def _():
        acc_ref[...] = acc_ref[...] + contrib

    @pl.when(i == n_it - 1)
    def _():
        cstride = xbuf_ref.shape[0]
        iota_e = lax.broadcasted_iota(jnp.int32, (cstride, E), 1)
        wcolv = jnp.sum(jnp.where(iota_e == e, wcol_ref[...], 0.0),
                        axis=1, keepdims=True)
        ybuf_ref[...] = jnp.where(wcolv > 0.0, acc_ref[...] * wcolv, 0.0)


def _run_ffn(xbuf, w1, w2, wcol, cstride):
    n_it = INTER // IT
    return pl.pallas_call(
        functools.partial(_ffn_kernel, n_it),
        grid=(E, n_it),
        in_specs=[
            pl.BlockSpec((cstride, H), lambda e, i: (e, 0)),
            pl.BlockSpec((1, IT, H), lambda e, i: (e, i, 0)),
            pl.BlockSpec((1, IT, H), lambda e, i: (e, i + INTER // IT, 0)),
            pl.BlockSpec((1, H, IT), lambda e, i: (e, 0, i)),
            pl.BlockSpec((cstride, E), lambda e, i: (0, 0)),
        ],
        out_specs=pl.BlockSpec((cstride, H), lambda e, i: (e, 0)),
        out_shape=jax.ShapeDtypeStruct((E * cstride, H), jnp.float32),
        scratch_shapes=[pltpu.VMEM((cstride, H), jnp.float32)],
    )(xbuf, w1, w1, w2, wcol)


def kernel(x, t, Wg1, bg1, Wg2, Wh1, Wh2, Wl1, Wl2, snr_threshold=0.5):
    B, N, C = x.shape
    T = B * N
    cap = int(1.25 * T / E)
    cstride = ((cap + 8) + 7) // 8 * 8

    tokens = x.reshape(T, C)
    wg1x = Wg1[:, :H]
    wg1s = jnp.pad(Wg1[:, H:], ((0, 0), (0, 2)))
    bg1c = bg1.reshape(H // 2, 1)

    wcol, tid = pl.pallas_call(
        functools.partial(_routing_kernel, cap, cstride),
        out_shape=(
            jax.ShapeDtypeStruct((cstride, E), jnp.float32),
            jax.ShapeDtypeStruct((cstride, E), jnp.int32),
        ),
    )(tokens, wg1x, wg1s, bg1c, Wg2)
    n_pad = (E * cstride + NW * 8 - 1) // (NW * 8) * (NW * 8)
    tok_pad = jnp.pad(tid.T.reshape(-1), (0, n_pad - E * cstride))
    xbuf = _sc_dispatch(tokens, tok_pad, n_pad)

    use_low = t.mean() >= snr_threshold
    ybuf = lax.cond(
        use_low,
        lambda xb, wc: _run_ffn(xb, Wl1, Wl2, wc, cstride),
        lambda xb, wc: _run_ffn(xb, Wh1, Wh2, wc, cstride),
        xbuf, wcol)

    out = pl.pallas_call(
        _combine_kernel,
        grid=(E,),
        in_specs=[
            pl.BlockSpec((cstride, H), lambda e: (e, 0)),
            pl.BlockSpec((cstride, E), lambda e: (0, 0)),
            pl.BlockSpec((cstride, E), lambda e: (0, 0)),
        ],
        out_specs=pl.BlockSpec((T, H), lambda e: (0, 0)),
        out_shape=jax.ShapeDtypeStruct((T, H), jnp.float32),
    )(ybuf, tid, wcol)

    y = out.reshape(B, N, C)
    return (y, jnp.zeros((), dtype=jnp.float32))
